# Initial kernel scaffold; baseline (speedup 1.0000x reference)
import jax
import jax.numpy as jnp
from jax.experimental import pallas as pl


def kernel(x, table):
    raise NotImplementedError("write your pallas kernel here")


# speedup vs baseline: 29.6434x; 29.6434x over previous
import functools
import numpy as np
import jax
import jax.numpy as jnp
from jax import lax
from jax.experimental import pallas as pl
from jax.experimental.pallas import tpu as pltpu
from jax.experimental.pallas import tpu_sc as plsc

_N_LEVELS = 16
_F = 2
_LOG2_T = 19
_T = 1 << _LOG2_T
_BASE = 16
_FINEST = 512
_N_PTS = 524288
_SCALE = np.exp(np.log(_FINEST / _BASE) / (_N_LEVELS - 1))
_RES = [int(np.floor(_BASE * _SCALE ** l)) for l in range(_N_LEVELS)]
_P1 = int(np.uint32(2654435761).astype(np.int32))
_P2 = int(np.uint32(805459861).astype(np.int32))

_NC = 2
_NS = 16
_NW = _NC * _NS
_C = 128
_D = 2 * _C
_G = _D // 16
_NSTR = (8 * _D) // 128
_PW = _N_PTS // _NW
_NCHUNK = _PW // _C


def _encode_body(xd_hbm, tab_hbm, out_hbm, xd_v, idx_v, w_v, feats_v, out_v,
                 sem):
  wid = lax.axis_index("s") * _NC + lax.axis_index("c")
  parity = lax.iota(jnp.int32, 16) & 1

  def chunk_body(ch, carry):
    gbase = pl.multiple_of(wid * _PW + ch * _C, _C)
    pltpu.sync_copy(xd_hbm.at[:, pl.ds(2 * gbase, _D)], xd_v)

    for l in range(_N_LEVELS):
      res = _RES[l]
      stride = res + 1
      dense = stride ** 3 <= _T

      def idx_body(i, c, l=l, res=res, stride=stride, dense=dense):
        off = pl.multiple_of(i * 16, 16)
        d0 = xd_v[0, pl.ds(off, 16)] * jnp.float32(res)
        d1 = xd_v[1, pl.ds(off, 16)] * jnp.float32(res)
        d2 = xd_v[2, pl.ds(off, 16)] * jnp.float32(res)
        i0 = d0.astype(jnp.int32)
        i1 = d1.astype(jnp.int32)
        i2 = d2.astype(jnp.int32)
        f0 = d0 - i0.astype(jnp.float32)
        f1 = d1 - i1.astype(jnp.float32)
        f2 = d2 - i2.astype(jnp.float32)
        w0 = (1.0 - f0, f0)
        w1 = (1.0 - f1, f1)
        w2 = (1.0 - f2, f2)
        if dense:
          t0 = (i0, i0 + 1)
          a1 = i1 * stride
          t1 = (a1, a1 + stride)
          a2 = i2 * (stride * stride)
          t2 = (a2, a2 + stride * stride)
        else:
          t0 = (i0, i0 + 1)
          h1 = i1 * jnp.int32(_P1)
          t1 = (h1, h1 + jnp.int32(_P1))
          h2 = i2 * jnp.int32(_P2)
          t2 = (h2, h2 + jnp.int32(_P2))
        base2 = jnp.int32(2 * l * _T) + parity
        for k in range(8):
          b0, b1, b2 = k & 1, (k >> 1) & 1, (k >> 2) & 1
          if dense:
            idx = t0[b0] + t1[b1] + t2[b2]
          else:
            idx = (t0[b0] ^ t1[b1] ^ t2[b2]) & jnp.int32(_T - 1)
          idx = idx + idx + base2
          w = w0[b0] * w1[b1] * w2[b2]
          q = k * _D + off
          r = q // 128
          cc = pl.multiple_of(q % 128, 16)
          idx_v[r, pl.ds(cc, 16)] = idx
          w_v[k, pl.ds(off, 16)] = w
        return c

      lax.fori_loop(0, _G, idx_body, 0)

      copies = [
          pltpu.make_async_copy(tab_hbm.at[idx_v.at[s]], feats_v.at[s], sem)
          for s in range(_NSTR)
      ]
      for cp in copies:
        cp.start()
      for cp in copies:
        cp.wait()

      def acc_body(i, c, l=l):
        off = pl.multiple_of(i * 16, 16)
        acc = jnp.zeros((16,), jnp.float32)
        for k in range(8):
          q = k * _D + off
          r = q // 128
          cc = pl.multiple_of(q % 128, 16)
          acc = acc + w_v[k, pl.ds(off, 16)] * feats_v[r, pl.ds(cc, 16)]
        out_v[l, pl.ds(off, 16)] = acc
        return c

      lax.fori_loop(0, _G, acc_body, 0)

    pltpu.sync_copy(out_v, out_hbm.at[:, pl.ds(2 * gbase, _D)])
    return carry

  lax.fori_loop(0, _NCHUNK, chunk_body, 0)


_encode = functools.partial(
    pl.kernel,
    out_type=jax.ShapeDtypeStruct((_N_LEVELS, 2 * _N_PTS), jnp.float32),
    mesh=plsc.VectorSubcoreMesh(core_axis_name="c", subcore_axis_name="s"),
    scratch_types=[
        pltpu.VMEM((3, _D), jnp.float32),
        pltpu.VMEM((_NSTR, 128), jnp.int32),
        pltpu.VMEM((8, _D), jnp.float32),
        pltpu.VMEM((_NSTR, 128), jnp.float32),
        pltpu.VMEM((_N_LEVELS, _D), jnp.float32),
        pltpu.SemaphoreType.DMA,
    ],
)(_encode_body)


@jax.jit
def kernel(x, table):
  xd = jnp.repeat(x.T, 2, axis=1)
  tab = table.reshape(_N_LEVELS * _T * _F)
  out = _encode(xd, tab)
  return out.reshape(_N_LEVELS, _N_PTS, _F).transpose(1, 0, 2).reshape(
      _N_PTS, _N_LEVELS * _F)


# speedup vs baseline: 33.4455x; 1.1283x over previous
import functools
import numpy as np
import jax
import jax.numpy as jnp
from jax import lax
from jax.experimental import pallas as pl
from jax.experimental.pallas import tpu as pltpu
from jax.experimental.pallas import tpu_sc as plsc

_N_LEVELS = 16
_F = 2
_LOG2_T = 19
_T = 1 << _LOG2_T
_BASE = 16
_FINEST = 512
_N_PTS = 524288
_SCALE = np.exp(np.log(_FINEST / _BASE) / (_N_LEVELS - 1))
_RES = [int(np.floor(_BASE * _SCALE ** l)) for l in range(_N_LEVELS)]
_P1 = int(np.uint32(2654435761).astype(np.int32))
_P2 = int(np.uint32(805459861).astype(np.int32))

_NC = 2
_NS = 16
_NW = _NC * _NS
_C = 128
_D = 2 * _C
_G = _D // 16
_NSTR = (8 * _D) // 128
_PW = _N_PTS // _NW
_NCHUNK = _PW // _C


def _encode_body(xd_hbm, tab_hbm, out_hbm, xd_v, idx_v, w_v, feats_v, out_v,
                 sem0, sem1):
  wid = lax.axis_index("s") * _NC + lax.axis_index("c")
  parity = lax.iota(jnp.int32, 16) & 1
  sems = (sem0, sem1)

  def idx_pass(l):
    res = _RES[l]
    stride = res + 1
    dense = stride ** 3 <= _T
    buf = l % 2

    def idx_body(i, c):
      off = pl.multiple_of(i * 16, 16)
      d0 = xd_v[0, pl.ds(off, 16)] * jnp.float32(res)
      d1 = xd_v[1, pl.ds(off, 16)] * jnp.float32(res)
      d2 = xd_v[2, pl.ds(off, 16)] * jnp.float32(res)
      i0 = d0.astype(jnp.int32)
      i1 = d1.astype(jnp.int32)
      i2 = d2.astype(jnp.int32)
      f0 = d0 - i0.astype(jnp.float32)
      f1 = d1 - i1.astype(jnp.float32)
      f2 = d2 - i2.astype(jnp.float32)
      w0 = (1.0 - f0, f0)
      w1 = (1.0 - f1, f1)
      w2 = (1.0 - f2, f2)
      if dense:
        t0 = (i0, i0 + 1)
        a1 = i1 * stride
        t1 = (a1, a1 + stride)
        a2 = i2 * (stride * stride)
        t2 = (a2, a2 + stride * stride)
      else:
        t0 = (i0, i0 + 1)
        h1 = i1 * jnp.int32(_P1)
        t1 = (h1, h1 + jnp.int32(_P1))
        h2 = i2 * jnp.int32(_P2)
        t2 = (h2, h2 + jnp.int32(_P2))
      base2 = jnp.int32(2 * l * _T) + parity
      for k in range(8):
        b0, b1, b2 = k & 1, (k >> 1) & 1, (k >> 2) & 1
        if dense:
          idx = t0[b0] + t1[b1] + t2[b2]
        else:
          idx = (t0[b0] ^ t1[b1] ^ t2[b2]) & jnp.int32(_T - 1)
        idx = idx + idx + base2
        w = w0[b0] * w1[b1] * w2[b2]
        q = k * _D + off
        r = q // 128
        cc = pl.multiple_of(q % 128, 16)
        idx_v[buf, r, pl.ds(cc, 16)] = idx
        w_v[buf, k, pl.ds(off, 16)] = w
      return c

    lax.fori_loop(0, _G, idx_body, 0)

  def fire(l):
    buf = l % 2
    for s in range(_NSTR):
      pltpu.make_async_copy(
          tab_hbm.at[idx_v.at[buf, s]], feats_v.at[buf, s], sems[buf]).start()

  def drain_acc(l):
    buf = l % 2
    for s in range(_NSTR):
      pltpu.make_async_copy(
          tab_hbm.at[idx_v.at[buf, s]], feats_v.at[buf, s], sems[buf]).wait()

    def acc_body(i, c):
      off = pl.multiple_of(i * 16, 16)
      acc = jnp.zeros((16,), jnp.float32)
      for k in range(8):
        q = k * _D + off
        r = q // 128
        cc = pl.multiple_of(q % 128, 16)
        acc = acc + w_v[buf, k, pl.ds(off, 16)] * feats_v[buf, r, pl.ds(cc, 16)]
      out_v[l, pl.ds(off, 16)] = acc
      return c

    lax.fori_loop(0, _G, acc_body, 0)

  def chunk_body(ch, carry):
    gbase = pl.multiple_of(wid * _PW + ch * _C, _C)
    pltpu.sync_copy(xd_hbm.at[:, pl.ds(2 * gbase, _D)], xd_v)

    idx_pass(0)
    fire(0)
    for l in range(1, _N_LEVELS):
      idx_pass(l)
      fire(l)
      drain_acc(l - 1)
    drain_acc(_N_LEVELS - 1)

    pltpu.sync_copy(out_v, out_hbm.at[:, pl.ds(2 * gbase, _D)])
    return carry

  lax.fori_loop(0, _NCHUNK, chunk_body, 0)


_encode = functools.partial(
    pl.kernel,
    out_type=jax.ShapeDtypeStruct((_N_LEVELS, 2 * _N_PTS), jnp.float32),
    mesh=plsc.VectorSubcoreMesh(core_axis_name="c", subcore_axis_name="s"),
    scratch_types=[
        pltpu.VMEM((3, _D), jnp.float32),
        pltpu.VMEM((2, _NSTR, 128), jnp.int32),
        pltpu.VMEM((2, 8, _D), jnp.float32),
        pltpu.VMEM((2, _NSTR, 128), jnp.float32),
        pltpu.VMEM((_N_LEVELS, _D), jnp.float32),
        pltpu.SemaphoreType.DMA,
        pltpu.SemaphoreType.DMA,
    ],
)(_encode_body)


@jax.jit
def kernel(x, table):
  xd = jnp.repeat(x.T, 2, axis=1)
  tab = table.reshape(_N_LEVELS * _T * _F)
  out = _encode(xd, tab)
  return out.reshape(_N_LEVELS, _N_PTS, _F).transpose(1, 0, 2).reshape(
      _N_PTS, _N_LEVELS * _F)
